# baseline (speedup 1.0000x reference)
import functools

import jax
import jax.numpy as jnp
from jax import lax
from jax.experimental import pallas as pl
from jax.experimental.pallas import tpu as pltpu
from jax.experimental.pallas import tpu_sc as plsc

N = 10000
E = 320000
H = 128
L = 3
G = 128
AF = 9
AV = 119
BV = 5
BT = BV ** 3

NC = 2
NS = 16
NW = NC * NS
EPW = E // NW
CH = 80
NCHUNK = EPW // CH
NPAD = 10240
RPT = NPAD // NS
ZR = 128

_INV_BN = 1.0 / (1.0 + 1e-5) ** 0.5


def _edge_body(h_hbm, pk_hbm, dst3_hbm, t_hbm, out_hbm,
               pk0, pk1, sv0, sv1, cv0, cv1, dst4,
               rows0, rows1, er0, er1, agg_sh, t_sh,
               semi0, semi1, semh0, semh1, seme0, seme1, sems0, sems1):
    c = lax.axis_index("c")
    s = lax.axis_index("s")
    w = c * NS + s
    ebase = w * EPW

    bufs = ((pk0, sv0, cv0, rows0, er0, semi0, semh0, seme0, sems0),
            (pk1, sv1, cv1, rows1, er1, semi1, semh1, seme1, sems1))

    def issue_idx(i, b):
        p = bufs[b][0]
        si = bufs[b][5]
        r4 = lax.rem(i, 4)
        pltpu.async_copy(pk_hbm.at[pl.ds(ebase + i * CH, CH)], p, si)
        pltpu.async_copy(dst3_hbm.at[w, i], dst4.at[pl.ds(r4 * 2, 2)], si)

    def wait_idx(b):
        p = bufs[b][0]
        si = bufs[b][5]
        pltpu.make_async_copy(pk_hbm.at[pl.ds(ebase, CH)], p, si).wait()
        pltpu.make_async_copy(dst3_hbm.at[w, 0], dst4.at[pl.ds(0, 2)],
                              si).wait()

    def unpack(b):
        p, sv, cv = bufs[b][0], bufs[b][1], bufs[b][2]

        def _u(j, _):
            sl = pl.ds(j * 16, 16)
            v = p[sl]
            sv[sl] = lax.shift_right_logical(v, 7)
            cv[sl] = lax.bitwise_and(v, 127)
            return 0

        lax.fori_loop(0, CH // 16, _u, 0)

    def issue_gathers(b):
        sv, cv, r, e = bufs[b][1], bufs[b][2], bufs[b][3], bufs[b][4]
        sh, se = bufs[b][6], bufs[b][7]
        pltpu.async_copy(h_hbm.at[sv], r, sh)
        pltpu.async_copy(t_hbm.at[cv], e, se)

    def wait_gathers(b):
        sv, cv, r, e = bufs[b][1], bufs[b][2], bufs[b][3], bufs[b][4]
        sh, se = bufs[b][6], bufs[b][7]
        pltpu.make_async_copy(h_hbm.at[sv], r, sh).wait()
        pltpu.make_async_copy(t_hbm.at[cv], e, se).wait()

    def compute_half(b, half):
        r, e = bufs[b][3], bufs[b][4]
        base = half * (CH // 2)

        def _rl(rr4, _):
            rr = base + rr4 * 4
            for u in range(4):
                for j in range(8):
                    sl = pl.ds(j * 16, 16)
                    r[rr + u, sl] = jnp.maximum(r[rr + u, sl] + e[rr + u, sl],
                                                0.0)
            return 0

        lax.fori_loop(0, CH // 8, _rl, 0)

    def scat_half(i, b, half):
        r, ss = bufs[b][3], bufs[b][8]
        pltpu.async_copy(r.at[pl.ds(half * (CH // 2), CH // 2)],
                         agg_sh.at[dst4.at[lax.rem(i, 4) * 2 + half]],
                         ss, add=True)

    def compute_scat(i, b):
        compute_half(b, 0)
        scat_half(i, b, 0)
        compute_half(b, 1)
        scat_half(i, b, 1)

    def wait_scat(b):
        r, ss = bufs[b][3], bufs[b][8]
        for half in range(2):
            pltpu.make_async_copy(r.at[pl.ds(half * (CH // 2), CH // 2)],
                                  agg_sh.at[dst4.at[half]], ss).wait()

    zero = jnp.zeros((16,), jnp.float32)

    def _zb(i, _):
        rows0[i // 8, pl.ds((i % 8) * 16, 16)] = zero
        return 0

    lax.fori_loop(0, CH * 8, _zb, 0)
    for k in range(RPT // CH):
        pltpu.sync_copy(rows0, agg_sh.at[pl.ds(s * RPT + k * CH, CH)])

    @pl.when(s == 0)
    def _():
        pltpu.sync_copy(t_hbm.at[pl.ds(0, CH)], rows1)
        pltpu.sync_copy(rows1, t_sh.at[pl.ds(0, CH)])
        pltpu.sync_copy(t_hbm.at[pl.ds(CH, BT - CH)], er0.at[pl.ds(0, BT - CH)])
        pltpu.sync_copy(er0.at[pl.ds(0, BT - CH)], t_sh.at[pl.ds(CH, BT - CH)])

    plsc.subcore_barrier()

    issue_idx(0, 0)
    issue_idx(1, 1)
    wait_idx(0)
    unpack(0)
    issue_gathers(0)

    def _pair(k, _):
        i0 = k * 2
        issue_idx(i0 + 2, 0)
        wait_gathers(0)
        wait_idx(1)
        unpack(1)

        @pl.when(k > 0)
        def _():
            wait_scat(1)

        issue_gathers(1)
        compute_scat(i0, 0)
        @pl.when(k < NCHUNK // 2 - 1)
        def _():
            issue_idx(i0 + 3, 1)

        wait_gathers(1)
        wait_idx(0)
        unpack(0)
        wait_scat(0)
        issue_gathers(0)
        compute_scat(i0 + 1, 1)
        return 0

    lax.fori_loop(0, NCHUNK // 2, _pair, 0)
    wait_gathers(0)
    compute_scat(NCHUNK - 1, 0)
    wait_scat(1)
    wait_scat(0)
    plsc.subcore_barrier()

    for k in range(RPT // CH):
        r0 = s * RPT + k * CH
        pltpu.sync_copy(agg_sh.at[pl.ds(r0, CH)], rows0)
        pltpu.sync_copy(rows0, out_hbm.at[pl.ds(c * NPAD + r0, CH)])


_edge_kernel = functools.partial(
    pl.kernel,
    out_type=jax.ShapeDtypeStruct((NC * NPAD, H), jnp.float32),
    mesh=plsc.VectorSubcoreMesh(core_axis_name="c", subcore_axis_name="s"),
    scratch_types=[
        pltpu.VMEM((CH,), jnp.int32),
        pltpu.VMEM((CH,), jnp.int32),
        pltpu.VMEM((CH,), jnp.int32),
        pltpu.VMEM((CH,), jnp.int32),
        pltpu.VMEM((CH,), jnp.int32),
        pltpu.VMEM((CH,), jnp.int32),
        pltpu.VMEM((8, CH // 2), jnp.int32),
        pltpu.VMEM((CH, H), jnp.float32),
        pltpu.VMEM((CH, H), jnp.float32),
        pltpu.VMEM((CH, H), jnp.float32),
        pltpu.VMEM((CH, H), jnp.float32),
        pltpu.VMEM_SHARED((NPAD, H), jnp.float32),
        pltpu.VMEM_SHARED((BT, H), jnp.float32),
        pltpu.SemaphoreType.DMA,
        pltpu.SemaphoreType.DMA,
        pltpu.SemaphoreType.DMA,
        pltpu.SemaphoreType.DMA,
        pltpu.SemaphoreType.DMA,
        pltpu.SemaphoreType.DMA,
        pltpu.SemaphoreType.DMA,
        pltpu.SemaphoreType.DMA,
    ],
)(_edge_body)


BN = 1000
EB = 2500


def _pack_body(src_ref, a0_ref, a1_ref, a2_ref, out_ref):
    out_ref[...] = (src_ref[...] * 128 + a0_ref[...] * 25 + a1_ref[...] * 5
                    + a2_ref[...])


def _pack_edges(src2, a02, a12, a22):
    bs = pl.BlockSpec((EB, H), lambda i: (0, 0))
    return pl.pallas_call(
        _pack_body,
        grid=(1,),
        in_specs=[bs, bs, bs, bs],
        out_specs=bs,
        out_shape=jax.ShapeDtypeStruct((EB, H), jnp.int32),
    )(src2, a02, a12, a22)


def _atom_body(x_ref, emb_ref, out_ref):
    xb = x_ref[...]
    lane = lax.broadcasted_iota(jnp.int32, (BN, H), 1)
    acc = jnp.zeros((BN, H), jnp.float32)
    for f in range(AF):
        oh = (xb[:, f:f + 1] == lane).astype(jnp.float32)
        acc = acc + jnp.dot(oh, emb_ref[f], preferred_element_type=jnp.float32)
    out_ref[...] = acc


def _atom_encode(x, emb_pad):
    return pl.pallas_call(
        _atom_body,
        grid=(N // BN,),
        in_specs=[
            pl.BlockSpec((BN, AF), lambda i: (i, 0)),
            pl.BlockSpec((AF, H, H), lambda i: (0, 0, 0)),
        ],
        out_specs=pl.BlockSpec((BN, H), lambda i: (i, 0)),
        out_shape=jax.ShapeDtypeStruct((N, H), jnp.float32),
    )(x, emb_pad)


def _mlp_body(eps_ref, h_ref, a0_ref, a1_ref, w1_ref, b1_ref, w2_ref, b2_ref,
              out_ref):
    pre = (1.0 + eps_ref[0, 0]) * h_ref[...] + a0_ref[...] + a1_ref[...]
    t = jnp.dot(pre, w1_ref[...], preferred_element_type=jnp.float32) + b1_ref[...]
    t = jnp.maximum(t * _INV_BN, 0.0)
    out_ref[...] = (jnp.dot(t, w2_ref[...], preferred_element_type=jnp.float32)
                    + b2_ref[...])


def _mlp(eps_l, h, agg0, agg1, w1, b1, w2, b2):
    return pl.pallas_call(
        _mlp_body,
        grid=(N // BN,),
        in_specs=[
            pl.BlockSpec((1, 1), lambda i: (0, 0)),
            pl.BlockSpec((BN, H), lambda i: (i, 0)),
            pl.BlockSpec((BN, H), lambda i: (i, 0)),
            pl.BlockSpec((BN, H), lambda i: (i, 0)),
            pl.BlockSpec((H, 2 * H), lambda i: (0, 0)),
            pl.BlockSpec((1, 2 * H), lambda i: (0, 0)),
            pl.BlockSpec((2 * H, H), lambda i: (0, 0)),
            pl.BlockSpec((1, H), lambda i: (0, 0)),
        ],
        out_specs=pl.BlockSpec((BN, H), lambda i: (i, 0)),
        out_shape=jax.ShapeDtypeStruct((N, H), jnp.float32),
    )(eps_l, h, agg0, agg1, w1, b1, w2, b2)


def _pool_body(eps_ref, h_ref, a0_ref, a1_ref, w1_ref, b1_ref, w2_ref, b2_ref,
               b_ref, wp_ref, bp_ref, beta_ref, mgf_ref, out_ref,
               s_acc, c_acc):
    i = pl.program_id(0)

    @pl.when(i == 0)
    def _():
        s_acc[...] = jnp.zeros_like(s_acc)
        c_acc[...] = jnp.zeros_like(c_acc)

    pre = (1.0 + eps_ref[0, 0]) * h_ref[...] + a0_ref[...] + a1_ref[...]
    t = jnp.dot(pre, w1_ref[...], preferred_element_type=jnp.float32) + b1_ref[...]
    t = jnp.maximum(t * _INV_BN, 0.0)
    nr = jnp.dot(t, w2_ref[...], preferred_element_type=jnp.float32) + b2_ref[...]

    gl = lax.broadcasted_iota(jnp.int32, (BN, G), 1)
    oh = (b_ref[...] == gl).astype(jnp.float32)
    s_acc[...] += lax.dot_general(oh, nr, (((0,), (0,)), ((), ())),
                                  preferred_element_type=jnp.float32)
    c_acc[...] += lax.dot_general(oh, jnp.ones((BN, H), jnp.float32),
                                  (((0,), (0,)), ((), ())),
                                  preferred_element_type=jnp.float32)

    @pl.when(i == pl.num_programs(0) - 1)
    def _():
        cnt = jnp.maximum(c_acc[:, 0:1], 1.0)
        sp = jnp.dot(s_acc[...], wp_ref[...], preferred_element_type=jnp.float32)
        pred = 1.0 / (1.0 + jnp.exp(-(sp / cnt + bp_ref[0, 0])))
        m = mgf_ref[...]
        mx = jnp.maximum(pred, m)
        ea = jnp.exp(beta_ref[0, 0] * (pred - mx))
        em = jnp.exp(beta_ref[0, 0] * (m - mx))
        out_ref[...] = (pred * ea + m * em) / (ea + em)


def _mlp_pool_head(eps_l, h, agg0, agg1, w1, b1, w2, b2,
                   batch2d, wp, bp, beta, mgf):
    return pl.pallas_call(
        _pool_body,
        grid=(N // BN,),
        in_specs=[
            pl.BlockSpec((1, 1), lambda i: (0, 0)),
            pl.BlockSpec((BN, H), lambda i: (i, 0)),
            pl.BlockSpec((BN, H), lambda i: (i, 0)),
            pl.BlockSpec((BN, H), lambda i: (i, 0)),
            pl.BlockSpec((H, 2 * H), lambda i: (0, 0)),
            pl.BlockSpec((1, 2 * H), lambda i: (0, 0)),
            pl.BlockSpec((2 * H, H), lambda i: (0, 0)),
            pl.BlockSpec((1, H), lambda i: (0, 0)),
            pl.BlockSpec((BN, 1), lambda i: (i, 0)),
            pl.BlockSpec((H, 1), lambda i: (0, 0)),
            pl.BlockSpec((1, 1), lambda i: (0, 0)),
            pl.BlockSpec((1, 1), lambda i: (0, 0)),
            pl.BlockSpec((G, 1), lambda i: (0, 0)),
        ],
        out_specs=pl.BlockSpec((G, 1), lambda i: (0, 0)),
        out_shape=jax.ShapeDtypeStruct((G, 1), jnp.float32),
        scratch_shapes=[
            pltpu.VMEM((G, H), jnp.float32),
            pltpu.VMEM((G, H), jnp.float32),
        ],
    )(eps_l, h, agg0, agg1, w1, b1, w2, b2, batch2d, wp, bp, beta, mgf)


def kernel(x, edge_index, edge_attr, batch, y, atom_emb, bond_embs, W1, b1,
           W2, b2, eps, Wp, bp, beta):
    x = x.astype(jnp.int32)
    src = edge_index[0].astype(jnp.int32)
    dst3 = edge_index[1].astype(jnp.int32).reshape(NW, NCHUNK, 2, CH // 2)
    ea = edge_attr.astype(jnp.int32)
    batch2d = batch.astype(jnp.int32).reshape(N, 1)
    packed = _pack_edges(src.reshape(EB, H), ea[:, 0].reshape(EB, H),
                         ea[:, 1].reshape(EB, H),
                         ea[:, 2].reshape(EB, H)).reshape(E)

    emb_pad = jnp.zeros((AF, H, H), jnp.float32).at[:, :AV, :].set(atom_emb)
    T = (bond_embs[:, 0][:, :, None, None, :]
         + bond_embs[:, 1][:, None, :, None, :]
         + bond_embs[:, 2][:, None, None, :, :]).reshape(L, BT, H)

    h = _atom_encode(x, emb_pad)
    for l in range(L - 1):
        agg = _edge_kernel(h, packed, dst3, T[l])
        h = _mlp(eps[l].reshape(1, 1), h, agg[:N], agg[NPAD:NPAD + N],
                 W1[l], b1[l].reshape(1, 2 * H), W2[l], b2[l].reshape(1, H))

    agg = _edge_kernel(h, packed, dst3, T[L - 1])
    mgf = y[:, 2].reshape(G, 1)
    return _mlp_pool_head(eps[L - 1].reshape(1, 1), h, agg[:N],
                          agg[NPAD:NPAD + N], W1[L - 1],
                          b1[L - 1].reshape(1, 2 * H), W2[L - 1],
                          b2[L - 1].reshape(1, H), batch2d, Wp,
                          bp.reshape(1, 1), beta.reshape(1, 1), mgf)


# speedup vs baseline: 1.4232x; 1.4232x over previous
import functools

import jax
import jax.numpy as jnp
from jax import lax
from jax.experimental import pallas as pl
from jax.experimental.pallas import tpu as pltpu
from jax.experimental.pallas import tpu_sc as plsc

N = 10000
E = 320000
H = 128
L = 3
G = 128
AF = 9
AV = 119
BV = 5
BT = BV ** 3

NC = 2
NS = 16
NW = NC * NS
EPW = E // NW
CH = 80
NCHUNK = EPW // CH
NPAD = 10240
RPT = NPAD // NS
ZR = 128

_INV_BN = 1.0 / (1.0 + 1e-5) ** 0.5


def _edge_body(h_hbm, pk_hbm, dst3_hbm, t_hbm, out_hbm,
               pk0, pk1, sv0, sv1, cv0, cv1, dst4,
               rows0, rows1, er0, er1, agg_sh, t_sh,
               semi0, semi1, semh0, semh1, seme0, seme1, sems0, sems1):
    c = lax.axis_index("c")
    s = lax.axis_index("s")
    w = c * NS + s
    ebase = w * EPW

    bufs = ((pk0, sv0, cv0, rows0, er0, semi0, semh0, seme0, sems0),
            (pk1, sv1, cv1, rows1, er1, semi1, semh1, seme1, sems1))

    def issue_idx(i, b):
        p = bufs[b][0]
        si = bufs[b][5]
        r4 = lax.rem(i, 4)
        pltpu.async_copy(pk_hbm.at[pl.ds(ebase + i * CH, CH)], p, si)
        pltpu.async_copy(dst3_hbm.at[w, i], dst4.at[pl.ds(r4 * 2, 2)], si)

    def wait_idx(b):
        p = bufs[b][0]
        si = bufs[b][5]
        pltpu.make_async_copy(pk_hbm.at[pl.ds(ebase, CH)], p, si).wait()
        pltpu.make_async_copy(dst3_hbm.at[w, 0], dst4.at[pl.ds(0, 2)],
                              si).wait()

    def unpack(b):
        p, sv, cv = bufs[b][0], bufs[b][1], bufs[b][2]

        def _u(j, _):
            sl = pl.ds(j * 16, 16)
            v = p[sl]
            sv[sl] = lax.shift_right_logical(v, 7)
            cv[sl] = lax.bitwise_and(v, 127)
            return 0

        lax.fori_loop(0, CH // 16, _u, 0)

    def issue_gather_h(b):
        sv, r, sh = bufs[b][1], bufs[b][3], bufs[b][6]
        pltpu.async_copy(h_hbm.at[sv], r, sh)

    def issue_gather_e(b):
        cv, e, se = bufs[b][2], bufs[b][4], bufs[b][7]
        pltpu.async_copy(t_sh.at[cv], e, se)

    def wait_gathers(b):
        sv, cv, r, e = bufs[b][1], bufs[b][2], bufs[b][3], bufs[b][4]
        sh, se = bufs[b][6], bufs[b][7]
        pltpu.make_async_copy(h_hbm.at[sv], r, sh).wait()
        pltpu.make_async_copy(t_sh.at[cv], e, se).wait()

    def compute_half(b, half):
        r, e = bufs[b][3], bufs[b][4]
        base = half * (CH // 2)

        def _rl(rr4, _):
            rr = base + rr4 * 4
            for u in range(4):
                for j in range(8):
                    sl = pl.ds(j * 16, 16)
                    e[rr + u, sl] = jnp.maximum(r[rr + u, sl] + e[rr + u, sl],
                                                0.0)
            return 0

        lax.fori_loop(0, CH // 8, _rl, 0)

    def scat_half(i, b, half):
        e, ss = bufs[b][4], bufs[b][8]
        pltpu.async_copy(e.at[pl.ds(half * (CH // 2), CH // 2)],
                         agg_sh.at[dst4.at[lax.rem(i, 4) * 2 + half]],
                         ss, add=True)

    def compute_scat(i, b):
        compute_half(b, 0)
        scat_half(i, b, 0)
        compute_half(b, 1)
        scat_half(i, b, 1)

    def wait_scat(b):
        e, ss = bufs[b][4], bufs[b][8]
        for half in range(2):
            pltpu.make_async_copy(e.at[pl.ds(half * (CH // 2), CH // 2)],
                                  agg_sh.at[dst4.at[half]], ss).wait()

    zero = jnp.zeros((16,), jnp.float32)

    def _zb(i, _):
        rows0[i // 8, pl.ds((i % 8) * 16, 16)] = zero
        return 0

    lax.fori_loop(0, CH * 8, _zb, 0)
    for k in range(RPT // CH):
        pltpu.sync_copy(rows0, agg_sh.at[pl.ds(s * RPT + k * CH, CH)])

    @pl.when(s == 0)
    def _():
        pltpu.sync_copy(t_hbm.at[pl.ds(0, CH)], rows1)
        pltpu.sync_copy(rows1, t_sh.at[pl.ds(0, CH)])
        pltpu.sync_copy(t_hbm.at[pl.ds(CH, BT - CH)], er0.at[pl.ds(0, BT - CH)])
        pltpu.sync_copy(er0.at[pl.ds(0, BT - CH)], t_sh.at[pl.ds(CH, BT - CH)])

    plsc.subcore_barrier()

    issue_idx(0, 0)
    issue_idx(1, 1)
    wait_idx(0)
    unpack(0)
    issue_gather_h(0)
    issue_gather_e(0)

    def _pair(k, _):
        i0 = k * 2
        issue_idx(i0 + 2, 0)
        wait_gathers(0)
        wait_idx(1)
        unpack(1)
        issue_gather_h(1)

        @pl.when(k > 0)
        def _():
            wait_scat(1)

        issue_gather_e(1)
        compute_scat(i0, 0)
        @pl.when(k < NCHUNK // 2 - 1)
        def _():
            issue_idx(i0 + 3, 1)

        wait_gathers(1)
        wait_idx(0)
        unpack(0)
        issue_gather_h(0)
        wait_scat(0)
        issue_gather_e(0)
        compute_scat(i0 + 1, 1)
        return 0

    lax.fori_loop(0, NCHUNK // 2, _pair, 0)
    wait_gathers(0)
    compute_scat(NCHUNK - 1, 0)
    wait_scat(1)
    wait_scat(0)
    plsc.subcore_barrier()

    for k in range(RPT // CH):
        r0 = s * RPT + k * CH
        pltpu.sync_copy(agg_sh.at[pl.ds(r0, CH)], rows0)
        pltpu.sync_copy(rows0, out_hbm.at[pl.ds(c * NPAD + r0, CH)])


_edge_kernel = functools.partial(
    pl.kernel,
    out_type=jax.ShapeDtypeStruct((NC * NPAD, H), jnp.float32),
    mesh=plsc.VectorSubcoreMesh(core_axis_name="c", subcore_axis_name="s"),
    scratch_types=[
        pltpu.VMEM((CH,), jnp.int32),
        pltpu.VMEM((CH,), jnp.int32),
        pltpu.VMEM((CH,), jnp.int32),
        pltpu.VMEM((CH,), jnp.int32),
        pltpu.VMEM((CH,), jnp.int32),
        pltpu.VMEM((CH,), jnp.int32),
        pltpu.VMEM((8, CH // 2), jnp.int32),
        pltpu.VMEM((CH, H), jnp.float32),
        pltpu.VMEM((CH, H), jnp.float32),
        pltpu.VMEM((CH, H), jnp.float32),
        pltpu.VMEM((CH, H), jnp.float32),
        pltpu.VMEM_SHARED((NPAD, H), jnp.float32),
        pltpu.VMEM_SHARED((BT, H), jnp.float32),
        pltpu.SemaphoreType.DMA,
        pltpu.SemaphoreType.DMA,
        pltpu.SemaphoreType.DMA,
        pltpu.SemaphoreType.DMA,
        pltpu.SemaphoreType.DMA,
        pltpu.SemaphoreType.DMA,
        pltpu.SemaphoreType.DMA,
        pltpu.SemaphoreType.DMA,
    ],
)(_edge_body)


BN = 1000
EB = 2500


def _pack_body(src_ref, a0_ref, a1_ref, a2_ref, out_ref):
    out_ref[...] = (src_ref[...] * 128 + a0_ref[...] * 25 + a1_ref[...] * 5
                    + a2_ref[...])


def _pack_edges(src2, a02, a12, a22):
    bs = pl.BlockSpec((EB, H), lambda i: (0, 0))
    return pl.pallas_call(
        _pack_body,
        grid=(1,),
        in_specs=[bs, bs, bs, bs],
        out_specs=bs,
        out_shape=jax.ShapeDtypeStruct((EB, H), jnp.int32),
    )(src2, a02, a12, a22)


def _atom_body(x_ref, emb_ref, out_ref):
    xb = x_ref[...]
    lane = lax.broadcasted_iota(jnp.int32, (BN, H), 1)
    acc = jnp.zeros((BN, H), jnp.float32)
    for f in range(AF):
        oh = (xb[:, f:f + 1] == lane).astype(jnp.float32)
        acc = acc + jnp.dot(oh, emb_ref[f], preferred_element_type=jnp.float32)
    out_ref[...] = acc


def _atom_encode(x, emb_pad):
    return pl.pallas_call(
        _atom_body,
        grid=(N // BN,),
        in_specs=[
            pl.BlockSpec((BN, AF), lambda i: (i, 0)),
            pl.BlockSpec((AF, H, H), lambda i: (0, 0, 0)),
        ],
        out_specs=pl.BlockSpec((BN, H), lambda i: (i, 0)),
        out_shape=jax.ShapeDtypeStruct((N, H), jnp.float32),
    )(x, emb_pad)


def _mlp_body(eps_ref, h_ref, a0_ref, a1_ref, w1_ref, b1_ref, w2_ref, b2_ref,
              out_ref):
    pre = (1.0 + eps_ref[0, 0]) * h_ref[...] + a0_ref[...] + a1_ref[...]
    t = jnp.dot(pre, w1_ref[...], preferred_element_type=jnp.float32) + b1_ref[...]
    t = jnp.maximum(t * _INV_BN, 0.0)
    out_ref[...] = (jnp.dot(t, w2_ref[...], preferred_element_type=jnp.float32)
                    + b2_ref[...])


def _mlp(eps_l, h, agg0, agg1, w1, b1, w2, b2):
    return pl.pallas_call(
        _mlp_body,
        grid=(N // BN,),
        in_specs=[
            pl.BlockSpec((1, 1), lambda i: (0, 0)),
            pl.BlockSpec((BN, H), lambda i: (i, 0)),
            pl.BlockSpec((BN, H), lambda i: (i, 0)),
            pl.BlockSpec((BN, H), lambda i: (i, 0)),
            pl.BlockSpec((H, 2 * H), lambda i: (0, 0)),
            pl.BlockSpec((1, 2 * H), lambda i: (0, 0)),
            pl.BlockSpec((2 * H, H), lambda i: (0, 0)),
            pl.BlockSpec((1, H), lambda i: (0, 0)),
        ],
        out_specs=pl.BlockSpec((BN, H), lambda i: (i, 0)),
        out_shape=jax.ShapeDtypeStruct((N, H), jnp.float32),
    )(eps_l, h, agg0, agg1, w1, b1, w2, b2)


def _pool_body(eps_ref, h_ref, a0_ref, a1_ref, w1_ref, b1_ref, w2_ref, b2_ref,
               b_ref, wp_ref, bp_ref, beta_ref, mgf_ref, out_ref,
               s_acc, c_acc):
    i = pl.program_id(0)

    @pl.when(i == 0)
    def _():
        s_acc[...] = jnp.zeros_like(s_acc)
        c_acc[...] = jnp.zeros_like(c_acc)

    pre = (1.0 + eps_ref[0, 0]) * h_ref[...] + a0_ref[...] + a1_ref[...]
    t = jnp.dot(pre, w1_ref[...], preferred_element_type=jnp.float32) + b1_ref[...]
    t = jnp.maximum(t * _INV_BN, 0.0)
    nr = jnp.dot(t, w2_ref[...], preferred_element_type=jnp.float32) + b2_ref[...]

    gl = lax.broadcasted_iota(jnp.int32, (BN, G), 1)
    oh = (b_ref[...] == gl).astype(jnp.float32)
    s_acc[...] += lax.dot_general(oh, nr, (((0,), (0,)), ((), ())),
                                  preferred_element_type=jnp.float32)
    c_acc[...] += lax.dot_general(oh, jnp.ones((BN, H), jnp.float32),
                                  (((0,), (0,)), ((), ())),
                                  preferred_element_type=jnp.float32)

    @pl.when(i == pl.num_programs(0) - 1)
    def _():
        cnt = jnp.maximum(c_acc[:, 0:1], 1.0)
        sp = jnp.dot(s_acc[...], wp_ref[...], preferred_element_type=jnp.float32)
        pred = 1.0 / (1.0 + jnp.exp(-(sp / cnt + bp_ref[0, 0])))
        m = mgf_ref[...]
        mx = jnp.maximum(pred, m)
        ea = jnp.exp(beta_ref[0, 0] * (pred - mx))
        em = jnp.exp(beta_ref[0, 0] * (m - mx))
        out_ref[...] = (pred * ea + m * em) / (ea + em)


def _mlp_pool_head(eps_l, h, agg0, agg1, w1, b1, w2, b2,
                   batch2d, wp, bp, beta, mgf):
    return pl.pallas_call(
        _pool_body,
        grid=(N // BN,),
        in_specs=[
            pl.BlockSpec((1, 1), lambda i: (0, 0)),
            pl.BlockSpec((BN, H), lambda i: (i, 0)),
            pl.BlockSpec((BN, H), lambda i: (i, 0)),
            pl.BlockSpec((BN, H), lambda i: (i, 0)),
            pl.BlockSpec((H, 2 * H), lambda i: (0, 0)),
            pl.BlockSpec((1, 2 * H), lambda i: (0, 0)),
            pl.BlockSpec((2 * H, H), lambda i: (0, 0)),
            pl.BlockSpec((1, H), lambda i: (0, 0)),
            pl.BlockSpec((BN, 1), lambda i: (i, 0)),
            pl.BlockSpec((H, 1), lambda i: (0, 0)),
            pl.BlockSpec((1, 1), lambda i: (0, 0)),
            pl.BlockSpec((1, 1), lambda i: (0, 0)),
            pl.BlockSpec((G, 1), lambda i: (0, 0)),
        ],
        out_specs=pl.BlockSpec((G, 1), lambda i: (0, 0)),
        out_shape=jax.ShapeDtypeStruct((G, 1), jnp.float32),
        scratch_shapes=[
            pltpu.VMEM((G, H), jnp.float32),
            pltpu.VMEM((G, H), jnp.float32),
        ],
    )(eps_l, h, agg0, agg1, w1, b1, w2, b2, batch2d, wp, bp, beta, mgf)


def kernel(x, edge_index, edge_attr, batch, y, atom_emb, bond_embs, W1, b1,
           W2, b2, eps, Wp, bp, beta):
    x = x.astype(jnp.int32)
    src = edge_index[0].astype(jnp.int32)
    dst3 = edge_index[1].astype(jnp.int32).reshape(NW, NCHUNK, 2, CH // 2)
    ea = edge_attr.astype(jnp.int32)
    batch2d = batch.astype(jnp.int32).reshape(N, 1)
    packed = _pack_edges(src.reshape(EB, H), ea[:, 0].reshape(EB, H),
                         ea[:, 1].reshape(EB, H),
                         ea[:, 2].reshape(EB, H)).reshape(E)

    emb_pad = jnp.zeros((AF, H, H), jnp.float32).at[:, :AV, :].set(atom_emb)
    T = (bond_embs[:, 0][:, :, None, None, :]
         + bond_embs[:, 1][:, None, :, None, :]
         + bond_embs[:, 2][:, None, None, :, :]).reshape(L, BT, H)

    h = _atom_encode(x, emb_pad)
    for l in range(L - 1):
        agg = _edge_kernel(h, packed, dst3, T[l])
        h = _mlp(eps[l].reshape(1, 1), h, agg[:N], agg[NPAD:NPAD + N],
                 W1[l], b1[l].reshape(1, 2 * H), W2[l], b2[l].reshape(1, H))

    agg = _edge_kernel(h, packed, dst3, T[L - 1])
    mgf = y[:, 2].reshape(G, 1)
    return _mlp_pool_head(eps[L - 1].reshape(1, 1), h, agg[:N],
                          agg[NPAD:NPAD + N], W1[L - 1],
                          b1[L - 1].reshape(1, 2 * H), W2[L - 1],
                          b2[L - 1].reshape(1, H), batch2d, Wp,
                          bp.reshape(1, 1), beta.reshape(1, 1), mgf)


# speedup vs baseline: 1.6188x; 1.1375x over previous
import functools

import jax
import jax.numpy as jnp
from jax import lax
from jax.experimental import pallas as pl
from jax.experimental.pallas import tpu as pltpu
from jax.experimental.pallas import tpu_sc as plsc

N = 10000
E = 320000
H = 128
L = 3
G = 128
AF = 9
AV = 119
BV = 5
BT = BV ** 3

NC = 2
NS = 16
NW = NC * NS
EPW = E // NW
CH = 80
NCHUNK = EPW // CH
NPAD = 10240
RPT = NPAD // NS
ZR = 128

_INV_BN = 1.0 / (1.0 + 1e-5) ** 0.5


def _edge_body(h_hbm, pk_hbm, dst3_hbm, t_hbm, out_hbm,
               pk0, pk1, sv0, sv1, cv0, cv1, dst4,
               rows0, rows1, er0, er1, agg_sh, t_sh,
               semi0, semi1, semh0, semh1, seme0, seme1, sems0, sems1):
    c = lax.axis_index("c")
    s = lax.axis_index("s")
    w = c * NS + s
    ebase = w * EPW

    bufs = ((pk0, sv0, cv0, rows0, er0, semi0, semh0, seme0, sems0),
            (pk1, sv1, cv1, rows1, er1, semi1, semh1, seme1, sems1))

    def issue_idx(i, b):
        p = bufs[b][0]
        si = bufs[b][5]
        r4 = lax.rem(i, 4)
        pltpu.async_copy(pk_hbm.at[pl.ds(ebase + i * CH, CH)], p, si)
        pltpu.async_copy(dst3_hbm.at[w, i], dst4.at[pl.ds(r4 * 2, 2)], si)

    def wait_idx(b):
        p = bufs[b][0]
        si = bufs[b][5]
        pltpu.make_async_copy(pk_hbm.at[pl.ds(ebase, CH)], p, si).wait()
        pltpu.make_async_copy(dst3_hbm.at[w, 0], dst4.at[pl.ds(0, 2)],
                              si).wait()

    def unpack(b):
        p, sv, cv = bufs[b][0], bufs[b][1], bufs[b][2]

        def _u(j, _):
            sl = pl.ds(j * 16, 16)
            v = p[sl]
            sv[sl] = lax.shift_right_logical(v, 7)
            cv[sl] = lax.bitwise_and(v, 127)
            return 0

        lax.fori_loop(0, CH // 16, _u, 0)

    def issue_gather_h(b):
        sv, r, sh = bufs[b][1], bufs[b][3], bufs[b][6]
        pltpu.async_copy(h_hbm.at[sv], r, sh)

    def issue_gather_e(b):
        cv, e, se = bufs[b][2], bufs[b][4], bufs[b][7]
        pltpu.async_copy(t_sh.at[cv], e, se)

    def wait_gathers(b):
        sv, cv, r, e = bufs[b][1], bufs[b][2], bufs[b][3], bufs[b][4]
        sh, se = bufs[b][6], bufs[b][7]
        pltpu.make_async_copy(h_hbm.at[sv], r, sh).wait()
        pltpu.make_async_copy(t_sh.at[cv], e, se).wait()

    def compute_half(b, half):
        r, e = bufs[b][3], bufs[b][4]
        base = half * (CH // 2)

        def _rl(rr4, _):
            rr = base + rr4 * 4
            for u in range(4):
                for j in range(8):
                    sl = pl.ds(j * 16, 16)
                    r[rr + u, sl] = jnp.maximum(r[rr + u, sl] + e[rr + u, sl],
                                                0.0)
            return 0

        lax.fori_loop(0, CH // 8, _rl, 0)

    def scat_half(i, b, half):
        r, ss = bufs[b][3], bufs[b][8]
        pltpu.async_copy(r.at[pl.ds(half * (CH // 2), CH // 2)],
                         agg_sh.at[dst4.at[lax.rem(i, 4) * 2 + half]],
                         ss, add=True)

    def compute_scat(i, b):
        compute_half(b, 0)
        scat_half(i, b, 0)
        compute_half(b, 1)
        scat_half(i, b, 1)

    def wait_scat(b):
        r, ss = bufs[b][3], bufs[b][8]
        for half in range(2):
            pltpu.make_async_copy(r.at[pl.ds(half * (CH // 2), CH // 2)],
                                  agg_sh.at[dst4.at[half]], ss).wait()

    zero = jnp.zeros((16,), jnp.float32)

    def _zb(i, _):
        rows0[i // 8, pl.ds((i % 8) * 16, 16)] = zero
        return 0

    lax.fori_loop(0, CH * 8, _zb, 0)
    for k in range(RPT // CH):
        pltpu.sync_copy(rows0, agg_sh.at[pl.ds(s * RPT + k * CH, CH)])

    @pl.when(s == 0)
    def _():
        pltpu.sync_copy(t_hbm.at[pl.ds(0, CH)], rows1)
        pltpu.sync_copy(rows1, t_sh.at[pl.ds(0, CH)])
        pltpu.sync_copy(t_hbm.at[pl.ds(CH, BT - CH)], er0.at[pl.ds(0, BT - CH)])
        pltpu.sync_copy(er0.at[pl.ds(0, BT - CH)], t_sh.at[pl.ds(CH, BT - CH)])

    plsc.subcore_barrier()

    issue_idx(0, 0)
    issue_idx(1, 1)
    wait_idx(0)
    unpack(0)
    issue_gather_h(0)
    issue_gather_e(0)

    def _pair(k, _):
        i0 = k * 2
        issue_idx(i0 + 2, 0)
        wait_gathers(0)
        wait_idx(1)
        unpack(1)

        @pl.when(k > 0)
        def _():
            wait_scat(1)

        issue_gather_h(1)
        issue_gather_e(1)
        compute_scat(i0, 0)
        @pl.when(k < NCHUNK // 2 - 1)
        def _():
            issue_idx(i0 + 3, 1)

        wait_gathers(1)
        wait_idx(0)
        unpack(0)
        wait_scat(0)
        issue_gather_h(0)
        issue_gather_e(0)
        compute_scat(i0 + 1, 1)
        return 0

    lax.fori_loop(0, NCHUNK // 2, _pair, 0)
    wait_gathers(0)
    compute_scat(NCHUNK - 1, 0)
    wait_scat(1)
    wait_scat(0)
    plsc.subcore_barrier()

    for k in range(RPT // CH):
        r0 = s * RPT + k * CH
        pltpu.sync_copy(agg_sh.at[pl.ds(r0, CH)], rows0)
        pltpu.sync_copy(rows0, out_hbm.at[pl.ds(c * NPAD + r0, CH)])


_edge_kernel = functools.partial(
    pl.kernel,
    out_type=jax.ShapeDtypeStruct((NC * NPAD, H), jnp.float32),
    mesh=plsc.VectorSubcoreMesh(core_axis_name="c", subcore_axis_name="s"),
    scratch_types=[
        pltpu.VMEM((CH,), jnp.int32),
        pltpu.VMEM((CH,), jnp.int32),
        pltpu.VMEM((CH,), jnp.int32),
        pltpu.VMEM((CH,), jnp.int32),
        pltpu.VMEM((CH,), jnp.int32),
        pltpu.VMEM((CH,), jnp.int32),
        pltpu.VMEM((8, CH // 2), jnp.int32),
        pltpu.VMEM((CH, H), jnp.float32),
        pltpu.VMEM((CH, H), jnp.float32),
        pltpu.VMEM((CH, H), jnp.float32),
        pltpu.VMEM((CH, H), jnp.float32),
        pltpu.VMEM_SHARED((NPAD, H), jnp.float32),
        pltpu.VMEM_SHARED((BT, H), jnp.float32),
        pltpu.SemaphoreType.DMA,
        pltpu.SemaphoreType.DMA,
        pltpu.SemaphoreType.DMA,
        pltpu.SemaphoreType.DMA,
        pltpu.SemaphoreType.DMA,
        pltpu.SemaphoreType.DMA,
        pltpu.SemaphoreType.DMA,
        pltpu.SemaphoreType.DMA,
    ],
)(_edge_body)


BN = 1000
EB = 2500


def _pack_body(src_ref, a0_ref, a1_ref, a2_ref, out_ref):
    out_ref[...] = (src_ref[...] * 128 + a0_ref[...] * 25 + a1_ref[...] * 5
                    + a2_ref[...])


def _pack_edges(src2, a02, a12, a22):
    bs = pl.BlockSpec((EB, H), lambda i: (0, 0))
    return pl.pallas_call(
        _pack_body,
        grid=(1,),
        in_specs=[bs, bs, bs, bs],
        out_specs=bs,
        out_shape=jax.ShapeDtypeStruct((EB, H), jnp.int32),
    )(src2, a02, a12, a22)


def _atom_body(x_ref, emb_ref, out_ref):
    xb = x_ref[...]
    lane = lax.broadcasted_iota(jnp.int32, (BN, H), 1)
    acc = jnp.zeros((BN, H), jnp.float32)
    for f in range(AF):
        oh = (xb[:, f:f + 1] == lane).astype(jnp.float32)
        acc = acc + jnp.dot(oh, emb_ref[f], preferred_element_type=jnp.float32)
    out_ref[...] = acc


def _atom_encode(x, emb_pad):
    return pl.pallas_call(
        _atom_body,
        grid=(N // BN,),
        in_specs=[
            pl.BlockSpec((BN, AF), lambda i: (i, 0)),
            pl.BlockSpec((AF, H, H), lambda i: (0, 0, 0)),
        ],
        out_specs=pl.BlockSpec((BN, H), lambda i: (i, 0)),
        out_shape=jax.ShapeDtypeStruct((N, H), jnp.float32),
    )(x, emb_pad)


def _mlp_body(eps_ref, h_ref, a0_ref, a1_ref, w1_ref, b1_ref, w2_ref, b2_ref,
              out_ref):
    pre = (1.0 + eps_ref[0, 0]) * h_ref[...] + a0_ref[...] + a1_ref[...]
    t = jnp.dot(pre, w1_ref[...], preferred_element_type=jnp.float32) + b1_ref[...]
    t = jnp.maximum(t * _INV_BN, 0.0)
    out_ref[...] = (jnp.dot(t, w2_ref[...], preferred_element_type=jnp.float32)
                    + b2_ref[...])


def _mlp(eps_l, h, agg0, agg1, w1, b1, w2, b2):
    return pl.pallas_call(
        _mlp_body,
        grid=(N // BN,),
        in_specs=[
            pl.BlockSpec((1, 1), lambda i: (0, 0)),
            pl.BlockSpec((BN, H), lambda i: (i, 0)),
            pl.BlockSpec((BN, H), lambda i: (i, 0)),
            pl.BlockSpec((BN, H), lambda i: (i, 0)),
            pl.BlockSpec((H, 2 * H), lambda i: (0, 0)),
            pl.BlockSpec((1, 2 * H), lambda i: (0, 0)),
            pl.BlockSpec((2 * H, H), lambda i: (0, 0)),
            pl.BlockSpec((1, H), lambda i: (0, 0)),
        ],
        out_specs=pl.BlockSpec((BN, H), lambda i: (i, 0)),
        out_shape=jax.ShapeDtypeStruct((N, H), jnp.float32),
    )(eps_l, h, agg0, agg1, w1, b1, w2, b2)


def _pool_body(eps_ref, h_ref, a0_ref, a1_ref, w1_ref, b1_ref, w2_ref, b2_ref,
               b_ref, wp_ref, bp_ref, beta_ref, mgf_ref, out_ref,
               s_acc, c_acc):
    i = pl.program_id(0)

    @pl.when(i == 0)
    def _():
        s_acc[...] = jnp.zeros_like(s_acc)
        c_acc[...] = jnp.zeros_like(c_acc)

    pre = (1.0 + eps_ref[0, 0]) * h_ref[...] + a0_ref[...] + a1_ref[...]
    t = jnp.dot(pre, w1_ref[...], preferred_element_type=jnp.float32) + b1_ref[...]
    t = jnp.maximum(t * _INV_BN, 0.0)
    nr = jnp.dot(t, w2_ref[...], preferred_element_type=jnp.float32) + b2_ref[...]

    gl = lax.broadcasted_iota(jnp.int32, (BN, G), 1)
    oh = (b_ref[...] == gl).astype(jnp.float32)
    s_acc[...] += lax.dot_general(oh, nr, (((0,), (0,)), ((), ())),
                                  preferred_element_type=jnp.float32)
    c_acc[...] += lax.dot_general(oh, jnp.ones((BN, H), jnp.float32),
                                  (((0,), (0,)), ((), ())),
                                  preferred_element_type=jnp.float32)

    @pl.when(i == pl.num_programs(0) - 1)
    def _():
        cnt = jnp.maximum(c_acc[:, 0:1], 1.0)
        sp = jnp.dot(s_acc[...], wp_ref[...], preferred_element_type=jnp.float32)
        pred = 1.0 / (1.0 + jnp.exp(-(sp / cnt + bp_ref[0, 0])))
        m = mgf_ref[...]
        mx = jnp.maximum(pred, m)
        ea = jnp.exp(beta_ref[0, 0] * (pred - mx))
        em = jnp.exp(beta_ref[0, 0] * (m - mx))
        out_ref[...] = (pred * ea + m * em) / (ea + em)


def _mlp_pool_head(eps_l, h, agg0, agg1, w1, b1, w2, b2,
                   batch2d, wp, bp, beta, mgf):
    return pl.pallas_call(
        _pool_body,
        grid=(N // BN,),
        in_specs=[
            pl.BlockSpec((1, 1), lambda i: (0, 0)),
            pl.BlockSpec((BN, H), lambda i: (i, 0)),
            pl.BlockSpec((BN, H), lambda i: (i, 0)),
            pl.BlockSpec((BN, H), lambda i: (i, 0)),
            pl.BlockSpec((H, 2 * H), lambda i: (0, 0)),
            pl.BlockSpec((1, 2 * H), lambda i: (0, 0)),
            pl.BlockSpec((2 * H, H), lambda i: (0, 0)),
            pl.BlockSpec((1, H), lambda i: (0, 0)),
            pl.BlockSpec((BN, 1), lambda i: (i, 0)),
            pl.BlockSpec((H, 1), lambda i: (0, 0)),
            pl.BlockSpec((1, 1), lambda i: (0, 0)),
            pl.BlockSpec((1, 1), lambda i: (0, 0)),
            pl.BlockSpec((G, 1), lambda i: (0, 0)),
        ],
        out_specs=pl.BlockSpec((G, 1), lambda i: (0, 0)),
        out_shape=jax.ShapeDtypeStruct((G, 1), jnp.float32),
        scratch_shapes=[
            pltpu.VMEM((G, H), jnp.float32),
            pltpu.VMEM((G, H), jnp.float32),
        ],
    )(eps_l, h, agg0, agg1, w1, b1, w2, b2, batch2d, wp, bp, beta, mgf)


def kernel(x, edge_index, edge_attr, batch, y, atom_emb, bond_embs, W1, b1,
           W2, b2, eps, Wp, bp, beta):
    x = x.astype(jnp.int32)
    src = edge_index[0].astype(jnp.int32)
    dst3 = edge_index[1].astype(jnp.int32).reshape(NW, NCHUNK, 2, CH // 2)
    ea = edge_attr.astype(jnp.int32)
    batch2d = batch.astype(jnp.int32).reshape(N, 1)
    packed = _pack_edges(src.reshape(EB, H), ea[:, 0].reshape(EB, H),
                         ea[:, 1].reshape(EB, H),
                         ea[:, 2].reshape(EB, H)).reshape(E)

    emb_pad = jnp.zeros((AF, H, H), jnp.float32).at[:, :AV, :].set(atom_emb)
    T = (bond_embs[:, 0][:, :, None, None, :]
         + bond_embs[:, 1][:, None, :, None, :]
         + bond_embs[:, 2][:, None, None, :, :]).reshape(L, BT, H)

    h = _atom_encode(x, emb_pad)
    for l in range(L - 1):
        agg = _edge_kernel(h, packed, dst3, T[l])
        h = _mlp(eps[l].reshape(1, 1), h, agg[:N], agg[NPAD:NPAD + N],
                 W1[l], b1[l].reshape(1, 2 * H), W2[l], b2[l].reshape(1, H))

    agg = _edge_kernel(h, packed, dst3, T[L - 1])
    mgf = y[:, 2].reshape(G, 1)
    return _mlp_pool_head(eps[L - 1].reshape(1, 1), h, agg[:N],
                          agg[NPAD:NPAD + N], W1[L - 1],
                          b1[L - 1].reshape(1, 2 * H), W2[L - 1],
                          b2[L - 1].reshape(1, H), batch2d, Wp,
                          bp.reshape(1, 1), beta.reshape(1, 1), mgf)
